# speedup vs baseline: 1.2629x; 1.2629x over previous
import functools

import jax
import jax.numpy as jnp
from jax import lax
from jax.experimental import pallas as pl
from jax.experimental.pallas import tpu as pltpu
from jax.experimental.pallas import tpu_sc as plsc

N = 10000
K = 32
F = 128
L = 16
NF = F // L

NC = 2
NS = 16
NW = NC * NS

CB = 4
ROWS_W = 320
NPAD = NW * ROWS_W
CHUNKS = ROWS_W // CB


def _sc_body(x_hbm, idx_hbm, out_hbm, idx_v, neigh_v, out_v, sem):
    wid = lax.axis_index("s") * NC + lax.axis_index("c")
    row0 = wid * ROWS_W

    def chunk(c, carry):
        r0 = row0 + c * CB
        pltpu.sync_copy(idx_hbm.at[pl.ds(r0 * K, CB * K)], idx_v)
        pltpu.async_copy(x_hbm.at[idx_v], neigh_v, sem).wait()
        inv_k = jnp.float32(1.0 / K)
        for d in range(CB):
            def kbody(k, c2, d=d):
                sums, maxs = c2
                base = d * K + k
                ns, nm = [], []
                for f in range(NF):
                    v = neigh_v[base, pl.ds(f * L, L)]
                    ns.append(sums[f] + v)
                    nm.append(jnp.maximum(maxs[f], v))
                return tuple(ns), tuple(nm)

            z = tuple(jnp.zeros((L,), jnp.float32) for _ in range(NF))
            ninf = tuple(jnp.full((L,), -jnp.inf, jnp.float32) for _ in range(NF))
            sums, maxs = lax.fori_loop(0, K, kbody, (z, ninf))
            for f in range(NF):
                out_v[d, pl.ds(f * L, L)] = sums[f] * inv_k
                out_v[d, pl.ds(F + f * L, L)] = maxs[f]
        pltpu.sync_copy(out_v, out_hbm.at[pl.ds(r0, CB)])
        return carry

    lax.fori_loop(0, CHUNKS, chunk, 0)


@jax.jit
def _run(x, idx_flat):
    mesh = plsc.VectorSubcoreMesh(
        core_axis_name="c", subcore_axis_name="s",
        num_cores=NC, num_subcores=NS,
    )
    kfn = pl.kernel(
        _sc_body,
        out_type=jax.ShapeDtypeStruct((NPAD, 2 * F), jnp.float32),
        mesh=mesh,
        scratch_types=[
            pltpu.VMEM((CB * K,), jnp.int32),
            pltpu.VMEM((CB * K, F), jnp.float32),
            pltpu.VMEM((CB, 2 * F), jnp.float32),
            pltpu.SemaphoreType.DMA,
        ],
    )
    return kfn(x, idx_flat)


def kernel(x, idxs):
    idx_flat = jnp.pad(idxs.reshape(-1), (0, (NPAD - N) * K))
    out = _run(x, idx_flat)
    return out[:N]


# speedup vs baseline: 1.4717x; 1.1653x over previous
import functools

import jax
import jax.numpy as jnp
from jax import lax
from jax.experimental import pallas as pl
from jax.experimental.pallas import tpu as pltpu
from jax.experimental.pallas import tpu_sc as plsc

N = 10000
K = 32
F = 128
L = 16
NF = F // L

NC = 2
NS = 16
NW = NC * NS

CB = 4
ROWS_W = 320
NPAD = NW * ROWS_W
CHUNKS = ROWS_W // CB
GI = CB * K
UK = 4


def _sc_body(x_hbm, idx_hbm, out_hbm, idx_all, neigh, out_v,
             gsem0, gsem1, osem0, osem1):
    wid = lax.axis_index("s") * NC + lax.axis_index("c")
    row0 = wid * ROWS_W

    pltpu.sync_copy(idx_hbm.at[pl.ds(wid * CHUNKS, CHUNKS)], idx_all)

    gsems = (gsem0, gsem1)
    osems = (osem0, osem1)

    def start_gather(slot, c):
        pltpu.make_async_copy(
            x_hbm.at[idx_all.at[c]], neigh.at[slot], gsems[slot]).start()

    def wait_gather(slot, c):
        pltpu.make_async_copy(
            x_hbm.at[idx_all.at[c]], neigh.at[slot], gsems[slot]).wait()

    def start_write(slot, c):
        pltpu.make_async_copy(
            out_v.at[slot], out_hbm.at[pl.ds(row0 + c * CB, CB)],
            osems[slot]).start()

    def wait_write(slot, c):
        pltpu.make_async_copy(
            out_v.at[slot], out_hbm.at[pl.ds(row0 + c * CB, CB)],
            osems[slot]).wait()

    inv_k = jnp.float32(1.0 / K)

    def compute(slot, c):
        for d in range(CB):
            def kbody(kk, c2, d=d):
                sums, maxs = c2
                for u in range(UK):
                    r = d * K + kk * UK + u
                    for f in range(NF):
                        v = neigh[slot, r, pl.ds(f * L, L)]
                        sums = sums[:f] + (sums[f] + v,) + sums[f + 1:]
                        maxs = maxs[:f] + (jnp.maximum(maxs[f], v),) + maxs[f + 1:]
                return sums, maxs

            z = tuple(jnp.zeros((L,), jnp.float32) for _ in range(NF))
            ninf = tuple(jnp.full((L,), -jnp.inf, jnp.float32) for _ in range(NF))
            sums, maxs = lax.fori_loop(0, K // UK, kbody, (z, ninf))
            for f in range(NF):
                out_v[slot, d, pl.ds(f * L, L)] = sums[f] * inv_k
                out_v[slot, d, pl.ds(F + f * L, L)] = maxs[f]

    start_gather(0, 0)
    start_gather(1, 1)

    def body(t, carry):
        for slot in range(2):
            c = 2 * t + slot
            wait_gather(slot, c)
            pl.when(t >= 1)(lambda slot=slot, c=c: wait_write(slot, c - 2))
            compute(slot, c)
            start_write(slot, c)
            pl.when(t <= (CHUNKS // 2 - 2))(
                lambda slot=slot, c=c: start_gather(slot, c + 2))
        return carry

    lax.fori_loop(0, CHUNKS // 2, body, 0)

    wait_write(0, CHUNKS - 2)
    wait_write(1, CHUNKS - 1)


@jax.jit
def _run(x, idx2d):
    mesh = plsc.VectorSubcoreMesh(
        core_axis_name="c", subcore_axis_name="s",
        num_cores=NC, num_subcores=NS,
    )
    kfn = pl.kernel(
        _sc_body,
        out_type=jax.ShapeDtypeStruct((NPAD, 2 * F), jnp.float32),
        mesh=mesh,
        scratch_types=[
            pltpu.VMEM((CHUNKS, GI), jnp.int32),
            pltpu.VMEM((2, GI, F), jnp.float32),
            pltpu.VMEM((2, CB, 2 * F), jnp.float32),
            pltpu.SemaphoreType.DMA,
            pltpu.SemaphoreType.DMA,
            pltpu.SemaphoreType.DMA,
            pltpu.SemaphoreType.DMA,
        ],
    )
    return kfn(x, idx2d)


def kernel(x, idxs):
    idx2d = jnp.pad(idxs.reshape(-1), (0, (NPAD - N) * K)).reshape(-1, GI)
    out = _run(x, idx2d)
    return out[:N]


# speedup vs baseline: 7.6608x; 5.2055x over previous
import functools

import jax
import jax.numpy as jnp
from jax import lax
from jax.experimental import pallas as pl
from jax.experimental.pallas import tpu as pltpu
from jax.experimental.pallas import tpu_sc as plsc

N = 10000
K = 32
F = 128
L = 16
NF = F // L

NC = 2
NS = 16
NW = NC * NS

CB = 4
ROWS_W = 320
NPAD = NW * ROWS_W
CHUNKS = ROWS_W // CB
GI = CB * K
UK = 4


def _sc_body(x_hbm, idx_hbm, out_hbm, x_sp, idx_all, neigh, out_v,
             gsem0, gsem1, osem0, osem1):
    sid = lax.axis_index("s")
    wid = sid * NC + lax.axis_index("c")
    row0 = wid * ROWS_W

    pl.when(sid == 0)(lambda: pltpu.sync_copy(x_hbm, x_sp))

    pltpu.sync_copy(idx_hbm.at[pl.ds(wid * CHUNKS, CHUNKS)], idx_all)
    plsc.subcore_barrier()

    gsems = (gsem0, gsem1)
    osems = (osem0, osem1)

    def start_gather(slot, c):
        pltpu.make_async_copy(
            x_sp.at[idx_all.at[c]], neigh.at[slot], gsems[slot]).start()

    def wait_gather(slot, c):
        pltpu.make_async_copy(
            x_sp.at[idx_all.at[c]], neigh.at[slot], gsems[slot]).wait()

    def start_write(slot, c):
        pltpu.make_async_copy(
            out_v.at[slot], out_hbm.at[pl.ds(row0 + c * CB, CB)],
            osems[slot]).start()

    def wait_write(slot, c):
        pltpu.make_async_copy(
            out_v.at[slot], out_hbm.at[pl.ds(row0 + c * CB, CB)],
            osems[slot]).wait()

    inv_k = jnp.float32(1.0 / K)

    def compute(slot, c):
        for d in range(CB):
            def kbody(kk, c2, d=d):
                sums, maxs = c2
                for u in range(UK):
                    r = d * K + kk * UK + u
                    for f in range(NF):
                        v = neigh[slot, r, pl.ds(f * L, L)]
                        sums = sums[:f] + (sums[f] + v,) + sums[f + 1:]
                        maxs = maxs[:f] + (jnp.maximum(maxs[f], v),) + maxs[f + 1:]
                return sums, maxs

            z = tuple(jnp.zeros((L,), jnp.float32) for _ in range(NF))
            ninf = tuple(jnp.full((L,), -jnp.inf, jnp.float32) for _ in range(NF))
            sums, maxs = lax.fori_loop(0, K // UK, kbody, (z, ninf))
            for f in range(NF):
                out_v[slot, d, pl.ds(f * L, L)] = sums[f] * inv_k
                out_v[slot, d, pl.ds(F + f * L, L)] = maxs[f]

    start_gather(0, 0)
    start_gather(1, 1)

    def body(t, carry):
        for slot in range(2):
            c = 2 * t + slot
            wait_gather(slot, c)
            pl.when(t >= 1)(lambda slot=slot, c=c: wait_write(slot, c - 2))
            compute(slot, c)
            start_write(slot, c)
            pl.when(t <= (CHUNKS // 2 - 2))(
                lambda slot=slot, c=c: start_gather(slot, c + 2))
        return carry

    lax.fori_loop(0, CHUNKS // 2, body, 0)

    wait_write(0, CHUNKS - 2)
    wait_write(1, CHUNKS - 1)


@jax.jit
def _run(x, idx2d):
    mesh = plsc.VectorSubcoreMesh(
        core_axis_name="c", subcore_axis_name="s",
        num_cores=NC, num_subcores=NS,
    )
    kfn = pl.kernel(
        _sc_body,
        out_type=jax.ShapeDtypeStruct((NPAD, 2 * F), jnp.float32),
        mesh=mesh,
        scratch_types=[
            pltpu.VMEM_SHARED((N, F), jnp.float32),
            pltpu.VMEM((CHUNKS, GI), jnp.int32),
            pltpu.VMEM((2, GI, F), jnp.float32),
            pltpu.VMEM((2, CB, 2 * F), jnp.float32),
            pltpu.SemaphoreType.DMA,
            pltpu.SemaphoreType.DMA,
            pltpu.SemaphoreType.DMA,
            pltpu.SemaphoreType.DMA,
        ],
    )
    return kfn(x, idx2d)


def kernel(x, idxs):
    idx2d = jnp.pad(idxs.reshape(-1), (0, (NPAD - N) * K)).reshape(-1, GI)
    out = _run(x, idx2d)
    return out[:N]
